# baseline (speedup 1.0000x reference)
import functools

import jax
import jax.numpy as jnp
from jax import lax
from jax.experimental import pallas as pl
from jax.experimental.pallas import tpu as pltpu
from jax.experimental.pallas import tpu_sc as plsc

B = 16384
D = 512
DW = D // 2
K = 20
ROWS_PER_SAMPLE = K + 2
NW = 32
NB = B // NW
G = 4
NG = NB // G
GROW = G * ROWS_PER_SAMPLE
NC = DW // 16

_MASK = -65536


def _sc_scores(idx_packed, w_packed):
    mesh = plsc.VectorSubcoreMesh(core_axis_name="c", subcore_axis_name="s")

    @functools.partial(
        pl.kernel,
        mesh=mesh,
        out_type=[
            jax.ShapeDtypeStruct((NW, NB // 8, 128), jnp.float32),
            jax.ShapeDtypeStruct((NW, NB // 8, 128), jnp.float32),
        ],
        scratch_types=[
            pltpu.VMEM((NG, GROW), jnp.int32),
            pltpu.VMEM((GROW, DW), jnp.int32),
            pltpu.VMEM((GROW, DW), jnp.int32),
            pltpu.VMEM((NB // 8, 128), jnp.float32),
            pltpu.VMEM((NB // 8, 128), jnp.float32),
            pltpu.SemaphoreType.DMA,
            pltpu.SemaphoreType.DMA,
        ],
    )
    def k(idx_hbm, w_hbm, pos_hbm, neg_hbm,
          idx_v, rows0, rows1, pos_v, neg_v, sem0, sem1):
        wid = lax.axis_index("s") * 2 + lax.axis_index("c")
        pltpu.sync_copy(idx_hbm.at[wid], idx_v)

        def lo_hi(x):
            lo = lax.bitcast_convert_type(x << 16, jnp.float32)
            hi = lax.bitcast_convert_type(
                x & jnp.full((16,), _MASK, jnp.int32), jnp.float32)
            return lo, hi

        def compute(g, rows_v):
            def sample_body(s, carry2):
                r0 = s * ROWS_PER_SAMPLE
                u = []
                for c in range(NC):
                    u.extend(lo_hi(rows_v[r0, pl.ds(16 * c, 16)]))

                def row_dot(r, acc):
                    a = acc
                    for c in range(NC):
                        lo, hi = lo_hi(rows_v[r, pl.ds(16 * c, 16)])
                        a = a + u[2 * c] * lo
                        a = a + u[2 * c + 1] * hi
                    return a

                pos = row_dot(r0 + 1, jnp.zeros((16,), jnp.float32))

                def neg_body(kk, acc):
                    return row_dot(r0 + 2 + kk, acc)

                neg = lax.fori_loop(
                    0, K, neg_body, jnp.zeros((16,), jnp.float32))
                sg = g * G + s
                pos_v[sg // 8, pl.ds((sg % 8) * 16, 16)] = pos
                neg_v[sg // 8, pl.ds((sg % 8) * 16, 16)] = neg
                return carry2

            lax.fori_loop(0, G, sample_body, 0)

        pltpu.async_copy(w_hbm.at[idx_v.at[0]], rows0, sem0)

        def pair_body(i, carry):
            g = 2 * i
            pltpu.make_async_copy(w_hbm.at[idx_v.at[g]], rows0, sem0).wait()
            pltpu.async_copy(w_hbm.at[idx_v.at[g + 1]], rows1, sem1)
            compute(g, rows0)
            pltpu.make_async_copy(w_hbm.at[idx_v.at[g + 1]], rows1, sem1).wait()

            @pl.when(i < NG // 2 - 1)
            def _():
                pltpu.async_copy(w_hbm.at[idx_v.at[g + 2]], rows0, sem0)

            compute(g + 1, rows1)
            return carry

        lax.fori_loop(0, NG // 2, pair_body, 0)
        pltpu.sync_copy(pos_v, pos_hbm.at[wid])
        pltpu.sync_copy(neg_v, neg_hbm.at[wid])

    return k(idx_packed, w_packed)


def _tc_loss(pos_part, neg_part):
    def body(pos_ref, neg_ref, out_ref):
        pos = jnp.sum(pos_ref[...], axis=1)
        neg = -jnp.sum(neg_ref[...], axis=1)
        def logsig(x):
            return jnp.minimum(x, 0.0) - jnp.log1p(jnp.exp(-jnp.abs(x)))
        total = jnp.sum(logsig(pos) + logsig(neg))
        out_ref[...] = jnp.reshape(-total / B, (1, 1))

    return pl.pallas_call(
        body,
        out_shape=jax.ShapeDtypeStruct((1, 1), jnp.float32),
    )(pos_part, neg_part)


def kernel(u_node, v_node, negative_nodes, W):
    idx = jnp.concatenate(
        [u_node.astype(jnp.int32),
         v_node.astype(jnp.int32),
         negative_nodes.astype(jnp.int32)], axis=1)
    idx_packed = idx.reshape(NW, NG, GROW)
    w_packed = lax.bitcast_convert_type(
        W.astype(jnp.bfloat16).reshape(-1, DW, 2), jnp.int32)
    pos_part, neg_part = _sc_scores(idx_packed, w_packed)
    loss = _tc_loss(pos_part.reshape(B, 16), neg_part.reshape(B, 16))
    return loss.reshape(())


# speedup vs baseline: 3.4487x; 3.4487x over previous
import functools

import jax
import jax.numpy as jnp
from jax import lax
from jax.experimental import pallas as pl
from jax.experimental.pallas import tpu as pltpu
from jax.experimental.pallas import tpu_sc as plsc

B = 16384
D = 512
DW = D // 2
K = 20
ROWS_PER_SAMPLE = K + 2
NW = 32
NB = B // NW
G = 4
NG = NB // G
GROW = G * ROWS_PER_SAMPLE
NC = DW // 16

_MASK = -65536


def _tc_pack(W):
    n = W.shape[0]
    blk = 1000

    def body(w_ref, out_ref):
        x = w_ref[...]
        lo = lax.bitcast_convert_type(
            x[:, :DW].astype(jnp.bfloat16).astype(jnp.float32), jnp.uint32)
        hi = lax.bitcast_convert_type(
            x[:, DW:].astype(jnp.bfloat16).astype(jnp.float32), jnp.uint32)
        word = hi | lax.shift_right_logical(lo, jnp.uint32(16))
        out_ref[...] = lax.bitcast_convert_type(word, jnp.int32)

    return pl.pallas_call(
        body,
        grid=(n // blk,),
        in_specs=[pl.BlockSpec((blk, D), lambda i: (i, 0))],
        out_specs=pl.BlockSpec((blk, DW), lambda i: (i, 0)),
        out_shape=jax.ShapeDtypeStruct((n, DW), jnp.int32),
    )(W)


def _sc_scores(idx_packed, w_packed):
    mesh = plsc.VectorSubcoreMesh(core_axis_name="c", subcore_axis_name="s")

    @functools.partial(
        pl.kernel,
        mesh=mesh,
        out_type=[
            jax.ShapeDtypeStruct((NW, NB // 8, 128), jnp.float32),
            jax.ShapeDtypeStruct((NW, NB // 8, 128), jnp.float32),
        ],
        scratch_types=[
            pltpu.VMEM((NG, GROW), jnp.int32),
            pltpu.VMEM((GROW, DW), jnp.int32),
            pltpu.VMEM((GROW, DW), jnp.int32),
            pltpu.VMEM((NB // 8, 128), jnp.float32),
            pltpu.VMEM((NB // 8, 128), jnp.float32),
            pltpu.SemaphoreType.DMA,
            pltpu.SemaphoreType.DMA,
        ],
    )
    def k(idx_hbm, w_hbm, pos_hbm, neg_hbm,
          idx_v, rows0, rows1, pos_v, neg_v, sem0, sem1):
        wid = lax.axis_index("s") * 2 + lax.axis_index("c")
        pltpu.sync_copy(idx_hbm.at[wid], idx_v)

        def lo_hi(x):
            lo = lax.bitcast_convert_type(x << 16, jnp.float32)
            hi = lax.bitcast_convert_type(
                x & jnp.full((16,), _MASK, jnp.int32), jnp.float32)
            return lo, hi

        def compute(g, rows_v):
            def sample_body(s, carry2):
                r0 = s * ROWS_PER_SAMPLE
                u = []
                for c in range(NC):
                    u.extend(lo_hi(rows_v[r0, pl.ds(16 * c, 16)]))

                def row_dot(r, accs):
                    a = list(accs)
                    for c in range(NC):
                        lo, hi = lo_hi(rows_v[r, pl.ds(16 * c, 16)])
                        i = (c % 2) * 2
                        a[i] = a[i] + u[2 * c] * lo
                        a[i + 1] = a[i + 1] + u[2 * c + 1] * hi
                    return tuple(a)

                zeros4 = tuple(jnp.zeros((16,), jnp.float32)
                               for _ in range(4))
                p = row_dot(r0 + 1, zeros4)
                pos = (p[0] + p[1]) + (p[2] + p[3])

                def neg_body(kk, accs):
                    return row_dot(r0 + 2 + kk, accs)

                nacc = lax.fori_loop(0, K, neg_body, zeros4)
                neg = (nacc[0] + nacc[1]) + (nacc[2] + nacc[3])
                sg = g * G + s
                pos_v[sg // 8, pl.ds((sg % 8) * 16, 16)] = pos
                neg_v[sg // 8, pl.ds((sg % 8) * 16, 16)] = neg
                return carry2

            lax.fori_loop(0, G, sample_body, 0)

        pltpu.async_copy(w_hbm.at[idx_v.at[0]], rows0, sem0)

        def pair_body(i, carry):
            g = 2 * i
            pltpu.make_async_copy(w_hbm.at[idx_v.at[g]], rows0, sem0).wait()
            pltpu.async_copy(w_hbm.at[idx_v.at[g + 1]], rows1, sem1)
            compute(g, rows0)
            pltpu.make_async_copy(w_hbm.at[idx_v.at[g + 1]], rows1, sem1).wait()

            @pl.when(i < NG // 2 - 1)
            def _():
                pltpu.async_copy(w_hbm.at[idx_v.at[g + 2]], rows0, sem0)

            compute(g + 1, rows1)
            return carry

        lax.fori_loop(0, NG // 2, pair_body, 0)
        pltpu.sync_copy(pos_v, pos_hbm.at[wid])
        pltpu.sync_copy(neg_v, neg_hbm.at[wid])

    return k(idx_packed, w_packed)


def _tc_loss(pos_part, neg_part):
    def body(pos_ref, neg_ref, out_ref):
        pos = jnp.sum(pos_ref[...], axis=1)
        neg = -jnp.sum(neg_ref[...], axis=1)
        def logsig(x):
            return jnp.minimum(x, 0.0) - jnp.log1p(jnp.exp(-jnp.abs(x)))
        total = jnp.sum(logsig(pos) + logsig(neg))
        out_ref[...] = jnp.reshape(-total / B, (1, 1))

    return pl.pallas_call(
        body,
        out_shape=jax.ShapeDtypeStruct((1, 1), jnp.float32),
    )(pos_part, neg_part)


def kernel(u_node, v_node, negative_nodes, W):
    idx = jnp.concatenate(
        [u_node.astype(jnp.int32),
         v_node.astype(jnp.int32),
         negative_nodes.astype(jnp.int32)], axis=1)
    idx_packed = idx.reshape(NW, NG, GROW)
    w_packed = _tc_pack(W)
    pos_part, neg_part = _sc_scores(idx_packed, w_packed)
    loss = _tc_loss(pos_part.reshape(B, 16), neg_part.reshape(B, 16))
    return loss.reshape(())
